# Initial kernel scaffold; baseline (speedup 1.0000x reference)
import jax
import jax.numpy as jnp
from jax.experimental import pallas as pl


def kernel(x, pos_embedding):
    raise NotImplementedError("write your pallas kernel here")


# speedup vs baseline: 2.2356x; 2.2356x over previous
import jax
import jax.numpy as jnp
from jax.experimental import pallas as pl
from jax.experimental.pallas import tpu as pltpu

BLOCK_S = 512


def _add_kernel(x_ref, pos_ref, out_ref):
    pos = pos_ref[...]
    out_ref[...] = x_ref[...] + pos[:, None, :]


def kernel(x, pos_embedding):
    seq_len, batch, d_model = x.shape
    grid = (seq_len // BLOCK_S,)
    return pl.pallas_call(
        _add_kernel,
        grid=grid,
        in_specs=[
            pl.BlockSpec((BLOCK_S, batch, d_model), lambda i: (i, 0, 0)),
            pl.BlockSpec((BLOCK_S, d_model), lambda i: (i, 0)),
        ],
        out_specs=pl.BlockSpec((BLOCK_S, batch, d_model), lambda i: (i, 0, 0)),
        out_shape=jax.ShapeDtypeStruct((seq_len, batch, d_model), x.dtype),
        compiler_params=pltpu.CompilerParams(
            dimension_semantics=("arbitrary",),
        ),
    )(x, pos_embedding)


# speedup vs baseline: 2.3423x; 1.0477x over previous
import jax
import jax.numpy as jnp
from jax.experimental import pallas as pl
from jax.experimental.pallas import tpu as pltpu

BLOCK_S = 1024


def _add_kernel(x_ref, pos_ref, out_ref):
    pos = pos_ref[...]
    out_ref[...] = x_ref[...] + pos[:, None, :]


def kernel(x, pos_embedding):
    seq_len, batch, d_model = x.shape
    grid = (seq_len // BLOCK_S,)
    return pl.pallas_call(
        _add_kernel,
        grid=grid,
        in_specs=[
            pl.BlockSpec((BLOCK_S, batch, d_model), lambda i: (i, 0, 0)),
            pl.BlockSpec((BLOCK_S, d_model), lambda i: (i, 0)),
        ],
        out_specs=pl.BlockSpec((BLOCK_S, batch, d_model), lambda i: (i, 0, 0)),
        out_shape=jax.ShapeDtypeStruct((seq_len, batch, d_model), x.dtype),
        compiler_params=pltpu.CompilerParams(
            dimension_semantics=("arbitrary",),
        ),
    )(x, pos_embedding)
